# baseline (speedup 1.0000x reference)
import functools

import jax
import jax.numpy as jnp
from jax import lax
from jax.experimental import pallas as pl
from jax.experimental.pallas import tpu as pltpu
from jax.experimental.pallas import tpu_sc as plsc

N = 10000
E = 320000
NFEAT = 128
HIDDEN = 64
NCLASS = 16

E1 = E + N
K = 128
NTILES = 32
NB0 = 52
NB1 = 112
NBMAX = max(NB0, NB1)
TOTB = 16 * (NB0 + NB1)
E2 = TOTB * K
N2 = 10240
ZR = N2 // 16
W80 = HIDDEN + 16

_f32 = jnp.float32
_i32 = jnp.int32


def _edge_body(eidx_hbm, as_hbm, ad_hbm, h_hbm, zn_hbm,
               acc_out,
               as_v, ad_v, sdvp, exv, si0, si1, di0, di1, hr0, hr1, sr0, sr1,
               acc_sp, semg0, semg1, sems0, sems1):
    cid = lax.axis_index("c")
    sid = lax.axis_index("s")

    pltpu.sync_copy(zn_hbm, acc_sp.at[pl.ds(sid * ZR, ZR)])

    @pl.when(cid == 0)
    def _():
        pltpu.sync_copy(eidx_hbm.at[pl.ds(sid * NB0, NB0)],
                        sdvp.at[pl.ds(0, NB0)])

    @pl.when(cid == 1)
    def _():
        pltpu.sync_copy(eidx_hbm.at[pl.ds(16 * NB0 + sid * NB1, NB1)],
                        sdvp.at[pl.ds(0, NB1)])

    pltpu.sync_copy(as_hbm, as_v)
    pltpu.sync_copy(ad_hbm, ad_v)

    plsc.subcore_barrier()

    nb = jnp.where(cid == 0, NB0, NB1)

    for g in range(K // 16):
        v = sdvp[0, pl.ds(g * 16, 16)]
        si0[0, pl.ds(g * 16, 16)] = v & 0xFFFF
    pltpu.async_copy(h_hbm.at[si0.at[0]], hr0, semg0)

    gd = lax.GatherDimensionNumbers(
        offset_dims=(), collapsed_slice_dims=(0,), start_index_map=(0,))
    iota16 = lax.broadcasted_iota(_i32, (16,), 0)
    col64 = jnp.full((16,), HIDDEN, _i32)
    z16 = jnp.zeros((16,), _f32)

    def _zpad(k, _):
        sr0[k, pl.ds(HIDDEN, 16)] = z16
        sr1[k, pl.ds(HIDDEN, 16)] = z16
        return _
    lax.fori_loop(0, K, _zpad, None)

    hrs = (hr0, hr1)
    srs = (sr0, sr1)
    sis = (si0, si1)
    dis = (di0, di1)
    semgs = (semg0, semg1)
    semss = (sems0, sems1)

    def _block2(i, _):
        for q in range(2):
            b = 2 * i + q
            hr, sr, dib = hrs[q], srs[q], dis[q]

            for g in range(K // 16):
                v = sdvp[b, pl.ds(g * 16, 16)]
                sidx = v & 0xFFFF
                didx = lax.shift_right_logical(v, 16)
                al = (plsc.load_gather(as_v, [sidx])
                      + plsc.load_gather(ad_v, [didx]))
                al = jnp.where(al > 0, al, 0.2 * al)
                exv[pl.ds(g * 16, 16)] = jnp.exp(al)

            pltpu.make_async_copy(h_hbm.at[pl.ds(0, K)], hr, semgs[q]).wait()

            @pl.when(b + 1 < nb)
            def _():
                for g in range(K // 16):
                    v = sdvp[b + 1, pl.ds(g * 16, 16)]
                    sis[1 - q][0, pl.ds(g * 16, 16)] = v & 0xFFFF
                pltpu.async_copy(
                    h_hbm.at[sis[1 - q].at[0]], hrs[1 - q], semgs[1 - q])

            @pl.when(b >= 2)
            def _():
                pltpu.make_async_copy(
                    acc_out.at[0, pl.ds(0, K)], sr, semss[q]).wait()

            for g in range(K // 16):
                v = sdvp[b, pl.ds(g * 16, 16)]
                dib[0, pl.ds(g * 16, 16)] = lax.shift_right_logical(v, 16)

            for g in range(K // 16):
                exg = exv[pl.ds(g * 16, 16)]
                plsc.store_scatter(sr, [g * 16 + iota16, col64], exg)
                for j in range(16):
                    m = lax.gather(
                        exg, jnp.full((16, 1), j, _i32), gd, slice_sizes=(1,),
                        mode=lax.GatherScatterMode.PROMISE_IN_BOUNDS)
                    row = g * 16 + j
                    for r in range(4):
                        sr[row, pl.ds(r * 16, 16)] = (
                            hr[row, pl.ds(r * 16, 16)] * m)

            pltpu.async_copy(sr, acc_sp.at[dib.at[0]], semss[q], add=True)
        return _

    lax.fori_loop(0, nb // 2, _block2, None)

    pltpu.make_async_copy(acc_out.at[0, pl.ds(0, K)], sr0, sems0).wait()
    pltpu.make_async_copy(acc_out.at[0, pl.ds(0, K)], sr1, sems1).wait()

    plsc.subcore_barrier()

    pltpu.sync_copy(acc_sp.at[pl.ds(sid * ZR, ZR)],
                    acc_out.at[cid, pl.ds(sid * ZR, ZR)])


_edge_sc = functools.partial(
    pl.kernel,
    mesh=plsc.VectorSubcoreMesh(core_axis_name="c", subcore_axis_name="s"),
    compiler_params=pltpu.CompilerParams(
        needs_layout_passes=False, use_tc_tiling_on_sc=False),
    out_type=jax.ShapeDtypeStruct((2, N2, W80), _f32),
    scratch_types=[
        pltpu.VMEM((N2,), _f32),
        pltpu.VMEM((N2,), _f32),
        pltpu.VMEM((NBMAX, K), _i32),
        pltpu.VMEM((K,), _f32),
        pltpu.VMEM((1, K), _i32),
        pltpu.VMEM((1, K), _i32),
        pltpu.VMEM((1, K), _i32),
        pltpu.VMEM((1, K), _i32),
        pltpu.VMEM((K, HIDDEN), _f32),
        pltpu.VMEM((K, HIDDEN), _f32),
        pltpu.VMEM((K, W80), _f32),
        pltpu.VMEM((K, W80), _f32),
        pltpu.VMEM_SHARED((N2, W80), _f32),
        pltpu.SemaphoreType.DMA,
        pltpu.SemaphoreType.DMA,
        pltpu.SemaphoreType.DMA,
        pltpu.SemaphoreType.DMA,
    ],
)(_edge_body)


_RB = 1280


def _tc_in_body(x_ref, w_ref, att_ref, h_ref, asd_ref):
    h = jnp.dot(x_ref[...], w_ref[...], preferred_element_type=_f32)
    h_ref[...] = h
    asd_ref[...] = jnp.dot(h, att_ref[...], preferred_element_type=_f32)


def _combine(acc_ref, b_ref):
    num = acc_ref[0, :, :HIDDEN] + acc_ref[1, :, :HIDDEN]
    den = acc_ref[0, :, HIDDEN:HIDDEN + 1] + acc_ref[1, :, HIDDEN:HIDDEN + 1]
    hin = num / (den + 1e-16) + b_ref[...]
    return jnp.where(hin > 0, hin, jnp.exp(hin) - 1.0)


def _tc_mid_body(acc_ref, b_ref, w_ref, att_ref, h_ref, asd_ref):
    hin = _combine(acc_ref, b_ref)
    h = jnp.dot(hin, w_ref[...], preferred_element_type=_f32)
    h_ref[...] = h
    asd_ref[...] = jnp.dot(h, att_ref[...], preferred_element_type=_f32)


def _tc_out_body(acc_ref, b_ref, w_ref, bo_ref, out_ref):
    hin = _combine(acc_ref, b_ref)
    out_ref[...] = (
        jnp.dot(hin, w_ref[...], preferred_element_type=_f32) + bo_ref[...])


def _full(shape):
    return pl.BlockSpec(shape, lambda i: tuple(0 for _ in shape))


def _tc_in(x_pad, w, att2):
    return pl.pallas_call(
        _tc_in_body,
        grid=(N2 // _RB,),
        in_specs=[
            pl.BlockSpec((_RB, x_pad.shape[1]), lambda i: (i, 0)),
            _full(w.shape),
            _full(att2.shape),
        ],
        out_specs=[
            pl.BlockSpec((_RB, HIDDEN), lambda i: (i, 0)),
            pl.BlockSpec((_RB, 2), lambda i: (i, 0)),
        ],
        out_shape=[
            jax.ShapeDtypeStruct((N2, HIDDEN), _f32),
            jax.ShapeDtypeStruct((N2, 2), _f32),
        ],
    )(x_pad, w, att2)


def _tc_mid(acc, b, w, att2):
    return pl.pallas_call(
        _tc_mid_body,
        grid=(N2 // _RB,),
        in_specs=[
            pl.BlockSpec((2, _RB, W80), lambda i: (0, i, 0)),
            _full(b.shape),
            _full(w.shape),
            _full(att2.shape),
        ],
        out_specs=[
            pl.BlockSpec((_RB, HIDDEN), lambda i: (i, 0)),
            pl.BlockSpec((_RB, 2), lambda i: (i, 0)),
        ],
        out_shape=[
            jax.ShapeDtypeStruct((N2, HIDDEN), _f32),
            jax.ShapeDtypeStruct((N2, 2), _f32),
        ],
    )(acc, b, w, att2)


def _tc_out(acc, b, w, bo):
    return pl.pallas_call(
        _tc_out_body,
        grid=(N2 // _RB,),
        in_specs=[
            pl.BlockSpec((2, _RB, W80), lambda i: (0, i, 0)),
            _full(b.shape),
            _full(w.shape),
            _full(bo.shape),
        ],
        out_specs=pl.BlockSpec((_RB, NCLASS), lambda i: (i, 0)),
        out_shape=jax.ShapeDtypeStruct((N2, NCLASS), _f32),
    )(acc, b, w, bo)


@jax.jit
def kernel(x, edge_index, W1, a_s1, a_d1, b1, W2, a_s2, a_d2, b2, W_out, b_out):
    loops = jnp.arange(N, dtype=_i32)
    src = jnp.concatenate(
        [edge_index[0].astype(_i32), loops, jnp.zeros((E2 - E1,), _i32)])
    pad_dst = N + jnp.arange(E2 - E1, dtype=_i32) % (N2 - N)
    dst = jnp.concatenate([edge_index[1].astype(_i32), loops, pad_dst])
    eidx = ((dst << 16) | src).reshape(TOTB, K)

    x_pad = jnp.pad(x, ((0, N2 - N), (0, 0)))
    zn = jnp.zeros((ZR, W80), _f32)

    att1 = jnp.stack([a_s1, a_d1], axis=1)
    att2 = jnp.stack([a_s2, a_d2], axis=1)

    h1, asd1 = _tc_in(x_pad, W1, att1)
    acc1 = _edge_sc(eidx, asd1[:, 0], asd1[:, 1], h1, zn)
    h2, asd2 = _tc_mid(acc1, b1.reshape(1, HIDDEN), W2, att2)
    acc2 = _edge_sc(eidx, asd2[:, 0], asd2[:, 1], h2, zn)
    out = _tc_out(acc2, b2.reshape(1, HIDDEN), W_out, b_out.reshape(1, NCLASS))
    return out[:N]


# speedup vs baseline: 1.1036x; 1.1036x over previous
import functools

import jax
import jax.numpy as jnp
from jax import lax
from jax.experimental import pallas as pl
from jax.experimental.pallas import tpu as pltpu
from jax.experimental.pallas import tpu_sc as plsc

N = 10000
E = 320000
NFEAT = 128
HIDDEN = 64
NCLASS = 16

E1 = E + N
K = 128
NTILES = 32
NB0 = 82
NB1 = 82
NBMAX = max(NB0, NB1)
TOTB = 16 * (NB0 + NB1)
E2 = TOTB * K
N2 = 10240
ZR = N2 // 16
W80 = HIDDEN + 16

_f32 = jnp.float32
_i32 = jnp.int32


def _edge_body(eidx_hbm, as_hbm, ad_hbm, h_hbm, zn_hbm,
               acc_out,
               as_v, ad_v, sdvp, exv, si0, si1, di0, di1, hr0, hr1, sr0, sr1,
               acc_sp, semg0, semg1, sems0, sems1):
    cid = lax.axis_index("c")
    sid = lax.axis_index("s")

    pltpu.sync_copy(zn_hbm, acc_sp.at[pl.ds(sid * ZR, ZR)])

    @pl.when(cid == 0)
    def _():
        pltpu.sync_copy(eidx_hbm.at[pl.ds(sid * NB0, NB0)],
                        sdvp.at[pl.ds(0, NB0)])

    @pl.when(cid == 1)
    def _():
        pltpu.sync_copy(eidx_hbm.at[pl.ds(16 * NB0 + sid * NB1, NB1)],
                        sdvp.at[pl.ds(0, NB1)])

    pltpu.sync_copy(as_hbm, as_v)
    pltpu.sync_copy(ad_hbm, ad_v)

    plsc.subcore_barrier()

    nb = jnp.where(cid == 0, NB0, NB1)

    for g in range(K // 16):
        v = sdvp[0, pl.ds(g * 16, 16)]
        si0[0, pl.ds(g * 16, 16)] = v & 0xFFFF
    pltpu.async_copy(h_hbm.at[si0.at[0]], hr0, semg0)

    gd = lax.GatherDimensionNumbers(
        offset_dims=(), collapsed_slice_dims=(0,), start_index_map=(0,))
    iota16 = lax.broadcasted_iota(_i32, (16,), 0)
    col64 = jnp.full((16,), HIDDEN, _i32)
    z16 = jnp.zeros((16,), _f32)

    def _zpad(k, _):
        sr0[k, pl.ds(HIDDEN, 16)] = z16
        sr1[k, pl.ds(HIDDEN, 16)] = z16
        return _
    lax.fori_loop(0, K, _zpad, None)

    hrs = (hr0, hr1)
    srs = (sr0, sr1)
    sis = (si0, si1)
    dis = (di0, di1)
    semgs = (semg0, semg1)
    semss = (sems0, sems1)

    def _block2(i, _):
        for q in range(2):
            b = 2 * i + q
            hr, sr, dib = hrs[q], srs[q], dis[q]

            for g in range(K // 16):
                v = sdvp[b, pl.ds(g * 16, 16)]
                sidx = v & 0xFFFF
                didx = lax.shift_right_logical(v, 16)
                al = (plsc.load_gather(as_v, [sidx])
                      + plsc.load_gather(ad_v, [didx]))
                al = jnp.where(al > 0, al, 0.2 * al)
                exv[pl.ds(g * 16, 16)] = jnp.exp(al)

            pltpu.make_async_copy(h_hbm.at[pl.ds(0, K)], hr, semgs[q]).wait()

            @pl.when(b + 1 < nb)
            def _():
                for g in range(K // 16):
                    v = sdvp[b + 1, pl.ds(g * 16, 16)]
                    sis[1 - q][0, pl.ds(g * 16, 16)] = v & 0xFFFF
                pltpu.async_copy(
                    h_hbm.at[sis[1 - q].at[0]], hrs[1 - q], semgs[1 - q])

            @pl.when(b >= 2)
            def _():
                pltpu.make_async_copy(
                    acc_out.at[0, pl.ds(0, K)], sr, semss[q]).wait()

            for g in range(K // 16):
                v = sdvp[b, pl.ds(g * 16, 16)]
                dib[0, pl.ds(g * 16, 16)] = lax.shift_right_logical(v, 16)

            for g in range(K // 16):
                exg = exv[pl.ds(g * 16, 16)]
                plsc.store_scatter(sr, [g * 16 + iota16, col64], exg)
                for j in range(16):
                    m = lax.gather(
                        exg, jnp.full((16, 1), j, _i32), gd, slice_sizes=(1,),
                        mode=lax.GatherScatterMode.PROMISE_IN_BOUNDS)
                    row = g * 16 + j
                    for r in range(2):
                        w = hr[row, pl.ds(r * 32, 32)]
                        a, bb = plsc.unpack(
                            w, format=plsc.PackFormat.INTERLEAVED)
                        sr[row, pl.ds(r * 32, 16)] = a * m
                        sr[row, pl.ds(r * 32 + 16, 16)] = bb * m

            pltpu.async_copy(sr, acc_sp.at[dib.at[0]], semss[q], add=True)
        return _

    lax.fori_loop(0, nb // 2, _block2, None)

    pltpu.make_async_copy(acc_out.at[0, pl.ds(0, K)], sr0, sems0).wait()
    pltpu.make_async_copy(acc_out.at[0, pl.ds(0, K)], sr1, sems1).wait()

    plsc.subcore_barrier()

    pltpu.sync_copy(acc_sp.at[pl.ds(sid * ZR, ZR)],
                    acc_out.at[cid, pl.ds(sid * ZR, ZR)])


_edge_sc = functools.partial(
    pl.kernel,
    mesh=plsc.VectorSubcoreMesh(core_axis_name="c", subcore_axis_name="s"),
    compiler_params=pltpu.CompilerParams(
        needs_layout_passes=False, use_tc_tiling_on_sc=False),
    out_type=jax.ShapeDtypeStruct((2, N2, W80), _f32),
    scratch_types=[
        pltpu.VMEM((N2,), _f32),
        pltpu.VMEM((N2,), _f32),
        pltpu.VMEM((NBMAX, K), _i32),
        pltpu.VMEM((K,), _f32),
        pltpu.VMEM((1, K), _i32),
        pltpu.VMEM((1, K), _i32),
        pltpu.VMEM((1, K), _i32),
        pltpu.VMEM((1, K), _i32),
        pltpu.VMEM((K, HIDDEN), jnp.bfloat16),
        pltpu.VMEM((K, HIDDEN), jnp.bfloat16),
        pltpu.VMEM((K, W80), _f32),
        pltpu.VMEM((K, W80), _f32),
        pltpu.VMEM_SHARED((N2, W80), _f32),
        pltpu.SemaphoreType.DMA,
        pltpu.SemaphoreType.DMA,
        pltpu.SemaphoreType.DMA,
        pltpu.SemaphoreType.DMA,
    ],
)(_edge_body)


_RB = 1280


def _perm_bf16(h):
    hv = jnp.swapaxes(h.reshape(_RB, 2, 2, 16), 2, 3)
    return hv.reshape(_RB, HIDDEN).astype(jnp.bfloat16)


def _tc_in_body(x_ref, w_ref, att_ref, h_ref, asd_ref):
    h = jnp.dot(x_ref[...], w_ref[...], preferred_element_type=_f32)
    h_ref[...] = _perm_bf16(h)
    asd_ref[...] = jnp.dot(h, att_ref[...], preferred_element_type=_f32)


def _combine(acc_ref, b_ref):
    num = acc_ref[0, :, :HIDDEN] + acc_ref[1, :, :HIDDEN]
    den = acc_ref[0, :, HIDDEN:HIDDEN + 1] + acc_ref[1, :, HIDDEN:HIDDEN + 1]
    hin = num / (den + 1e-16) + b_ref[...]
    return jnp.where(hin > 0, hin, jnp.exp(hin) - 1.0)


def _tc_mid_body(acc_ref, b_ref, w_ref, att_ref, h_ref, asd_ref):
    hin = _combine(acc_ref, b_ref)
    h = jnp.dot(hin, w_ref[...], preferred_element_type=_f32)
    h_ref[...] = _perm_bf16(h)
    asd_ref[...] = jnp.dot(h, att_ref[...], preferred_element_type=_f32)


def _tc_out_body(acc_ref, b_ref, w_ref, bo_ref, out_ref):
    hin = _combine(acc_ref, b_ref)
    out_ref[...] = (
        jnp.dot(hin, w_ref[...], preferred_element_type=_f32) + bo_ref[...])


def _full(shape):
    return pl.BlockSpec(shape, lambda i: tuple(0 for _ in shape))


def _tc_in(x_pad, w, att2):
    return pl.pallas_call(
        _tc_in_body,
        grid=(N2 // _RB,),
        in_specs=[
            pl.BlockSpec((_RB, x_pad.shape[1]), lambda i: (i, 0)),
            _full(w.shape),
            _full(att2.shape),
        ],
        out_specs=[
            pl.BlockSpec((_RB, HIDDEN), lambda i: (i, 0)),
            pl.BlockSpec((_RB, 2), lambda i: (i, 0)),
        ],
        out_shape=[
            jax.ShapeDtypeStruct((N2, HIDDEN), jnp.bfloat16),
            jax.ShapeDtypeStruct((N2, 2), _f32),
        ],
    )(x_pad, w, att2)


def _tc_mid(acc, b, w, att2):
    return pl.pallas_call(
        _tc_mid_body,
        grid=(N2 // _RB,),
        in_specs=[
            pl.BlockSpec((2, _RB, W80), lambda i: (0, i, 0)),
            _full(b.shape),
            _full(w.shape),
            _full(att2.shape),
        ],
        out_specs=[
            pl.BlockSpec((_RB, HIDDEN), lambda i: (i, 0)),
            pl.BlockSpec((_RB, 2), lambda i: (i, 0)),
        ],
        out_shape=[
            jax.ShapeDtypeStruct((N2, HIDDEN), jnp.bfloat16),
            jax.ShapeDtypeStruct((N2, 2), _f32),
        ],
    )(acc, b, w, att2)


def _tc_out(acc, b, w, bo):
    return pl.pallas_call(
        _tc_out_body,
        grid=(N2 // _RB,),
        in_specs=[
            pl.BlockSpec((2, _RB, W80), lambda i: (0, i, 0)),
            _full(b.shape),
            _full(w.shape),
            _full(bo.shape),
        ],
        out_specs=pl.BlockSpec((_RB, NCLASS), lambda i: (i, 0)),
        out_shape=jax.ShapeDtypeStruct((N2, NCLASS), _f32),
    )(acc, b, w, bo)


@jax.jit
def kernel(x, edge_index, W1, a_s1, a_d1, b1, W2, a_s2, a_d2, b2, W_out, b_out):
    loops = jnp.arange(N, dtype=_i32)
    src = jnp.concatenate(
        [edge_index[0].astype(_i32), loops, jnp.zeros((E2 - E1,), _i32)])
    pad_dst = N + jnp.arange(E2 - E1, dtype=_i32) % (N2 - N)
    dst = jnp.concatenate([edge_index[1].astype(_i32), loops, pad_dst])
    eidx = ((dst << 16) | src).reshape(TOTB, K)

    x_pad = jnp.pad(x, ((0, N2 - N), (0, 0)))
    zn = jnp.zeros((ZR, W80), _f32)

    att1 = jnp.stack([a_s1, a_d1], axis=1)
    att2 = jnp.stack([a_s2, a_d2], axis=1)

    h1, asd1 = _tc_in(x_pad, W1, att1)
    acc1 = _edge_sc(eidx, asd1[:, 0], asd1[:, 1], h1, zn)
    h2, asd2 = _tc_mid(acc1, b1.reshape(1, HIDDEN), W2, att2)
    acc2 = _edge_sc(eidx, asd2[:, 0], asd2[:, 1], h2, zn)
    out = _tc_out(acc2, b2.reshape(1, HIDDEN), W_out, b_out.reshape(1, NCLASS))
    return out[:N]


# speedup vs baseline: 1.5336x; 1.3897x over previous
import functools

import jax
import jax.numpy as jnp
from jax import lax
from jax.experimental import pallas as pl
from jax.experimental.pallas import tpu as pltpu
from jax.experimental.pallas import tpu_sc as plsc

N = 10000
E = 320000
NFEAT = 128
HIDDEN = 64
NCLASS = 16

E1 = E + N
K = 128
NTILES = 32
NB0 = 82
NB1 = 82
NBMAX = max(NB0, NB1)
TOTB = 16 * (NB0 + NB1)
E2 = TOTB * K
N2 = 10240
ZR = N2 // 16
W80 = HIDDEN + 16

_f32 = jnp.float32
_i32 = jnp.int32


def _edge_body(eidx_hbm, as_hbm, ad_hbm, h_hbm, zn_hbm,
               acc_out,
               as_v, ad_v, sdvp, exv, si0, si1, di0, di1, hr0, hr1, sr0, sr1,
               acc_sp, semg0, semg1, sems0, sems1):
    cid = lax.axis_index("c")
    sid = lax.axis_index("s")

    pltpu.sync_copy(zn_hbm, acc_sp.at[pl.ds(sid * ZR, ZR)])

    @pl.when(cid == 0)
    def _():
        pltpu.sync_copy(eidx_hbm.at[pl.ds(sid * NB0, NB0)],
                        sdvp.at[pl.ds(0, NB0)])

    @pl.when(cid == 1)
    def _():
        pltpu.sync_copy(eidx_hbm.at[pl.ds(16 * NB0 + sid * NB1, NB1)],
                        sdvp.at[pl.ds(0, NB1)])

    pltpu.sync_copy(as_hbm, as_v)
    pltpu.sync_copy(ad_hbm, ad_v)

    plsc.subcore_barrier()

    nb = jnp.where(cid == 0, NB0, NB1)

    for g in range(K // 16):
        v = sdvp[0, pl.ds(g * 16, 16)]
        si0[0, pl.ds(g * 16, 16)] = v & 0xFFFF
    pltpu.async_copy(h_hbm.at[si0.at[0]], hr0, semg0)

    gd = lax.GatherDimensionNumbers(
        offset_dims=(), collapsed_slice_dims=(0,), start_index_map=(0,))
    iota16 = lax.broadcasted_iota(_i32, (16,), 0)
    col64 = jnp.full((16,), HIDDEN, _i32)
    z16 = jnp.zeros((16,), _f32)

    def _zpad(k, _):
        sr0[k, pl.ds(HIDDEN, 16)] = z16
        sr1[k, pl.ds(HIDDEN, 16)] = z16
        return _
    lax.fori_loop(0, K, _zpad, None)

    hrs = (hr0, hr1)
    srs = (sr0, sr1)
    sis = (si0, si1)
    dis = (di0, di1)
    semgs = (semg0, semg1)
    semss = (sems0, sems1)

    def _block2(i, _):
        for q in range(2):
            b = 2 * i + q
            hr, sr, dib = hrs[q], srs[q], dis[q]

            for g in range(K // 16):
                v = sdvp[b, pl.ds(g * 16, 16)]
                sidx = v & 0xFFFF
                didx = lax.shift_right_logical(v, 16)
                al = (plsc.load_gather(as_v, [sidx])
                      + plsc.load_gather(ad_v, [didx]))
                al = jnp.where(al > 0, al, 0.2 * al)
                exv[pl.ds(g * 16, 16)] = jnp.exp(al)

            pltpu.make_async_copy(h_hbm.at[pl.ds(0, K)], hr, semgs[q]).wait()

            @pl.when(b + 1 < nb)
            def _():
                for g in range(K // 16):
                    v = sdvp[b + 1, pl.ds(g * 16, 16)]
                    sis[1 - q][0, pl.ds(g * 16, 16)] = v & 0xFFFF
                pltpu.async_copy(
                    h_hbm.at[sis[1 - q].at[0]], hrs[1 - q], semgs[1 - q])

            @pl.when(b >= 2)
            def _():
                pltpu.make_async_copy(
                    acc_out.at[0, pl.ds(0, K)], sr, semss[q]).wait()

            for g in range(K // 16):
                v = sdvp[b, pl.ds(g * 16, 16)]
                dib[0, pl.ds(g * 16, 16)] = lax.shift_right_logical(v, 16)

            for g in range(K // 16):
                exg = exv[pl.ds(g * 16, 16)]
                plsc.store_scatter(sr, [g * 16 + iota16, col64], exg)
                for j in range(16):
                    m = lax.gather(
                        exg, jnp.full((16, 1), j, _i32), gd, slice_sizes=(1,),
                        mode=lax.GatherScatterMode.PROMISE_IN_BOUNDS)
                    row = g * 16 + j
                    for r in range(2):
                        w = hr[row, pl.ds(r * 32, 32)]
                        a, bb = plsc.unpack(
                            w, format=plsc.PackFormat.INTERLEAVED)
                        sr[row, pl.ds(r * 32, 16)] = a * m
                        sr[row, pl.ds(r * 32 + 16, 16)] = bb * m

            pltpu.async_copy(sr, acc_sp.at[dib.at[0]], semss[q], add=True)
        return _

    lax.fori_loop(0, nb // 2, _block2, None)

    pltpu.make_async_copy(acc_out.at[0, pl.ds(0, K)], sr0, sems0).wait()
    pltpu.make_async_copy(acc_out.at[0, pl.ds(0, K)], sr1, sems1).wait()

    plsc.subcore_barrier()

    pltpu.sync_copy(acc_sp.at[pl.ds(sid * ZR, ZR)],
                    acc_out.at[cid, pl.ds(sid * ZR, ZR)])


_edge_sc = functools.partial(
    pl.kernel,
    mesh=plsc.VectorSubcoreMesh(core_axis_name="c", subcore_axis_name="s"),
    compiler_params=pltpu.CompilerParams(
        needs_layout_passes=False, use_tc_tiling_on_sc=False),
    out_type=jax.ShapeDtypeStruct((2, N2, W80), _f32),
    scratch_types=[
        pltpu.VMEM((N2,), _f32),
        pltpu.VMEM((N2,), _f32),
        pltpu.VMEM((NBMAX, K), _i32),
        pltpu.VMEM((K,), _f32),
        pltpu.VMEM((1, K), _i32),
        pltpu.VMEM((1, K), _i32),
        pltpu.VMEM((1, K), _i32),
        pltpu.VMEM((1, K), _i32),
        pltpu.VMEM((K, HIDDEN), jnp.bfloat16),
        pltpu.VMEM((K, HIDDEN), jnp.bfloat16),
        pltpu.VMEM((K, W80), _f32),
        pltpu.VMEM((K, W80), _f32),
        pltpu.VMEM_SHARED((N2, W80), _f32),
        pltpu.SemaphoreType.DMA,
        pltpu.SemaphoreType.DMA,
        pltpu.SemaphoreType.DMA,
        pltpu.SemaphoreType.DMA,
    ],
)(_edge_body)


_RB = 1280


def _tc_in_body(x_ref, w_ref, att_ref, h_ref, asd_ref):
    h = jnp.dot(x_ref[...], w_ref[...], preferred_element_type=_f32)
    h_ref[...] = h.astype(jnp.bfloat16)
    asd_ref[...] = jnp.dot(h, att_ref[...], preferred_element_type=_f32)


def _combine(acc_ref, b_ref):
    num = acc_ref[0, :, :HIDDEN] + acc_ref[1, :, :HIDDEN]
    den = acc_ref[0, :, HIDDEN:HIDDEN + 1] + acc_ref[1, :, HIDDEN:HIDDEN + 1]
    hin = num / (den + 1e-16) + b_ref[...]
    return jnp.where(hin > 0, hin, jnp.exp(hin) - 1.0)


def _tc_mid_body(acc_ref, b_ref, w_ref, att_ref, h_ref, asd_ref):
    hin = _combine(acc_ref, b_ref)
    h = jnp.dot(hin, w_ref[...], preferred_element_type=_f32)
    h_ref[...] = h.astype(jnp.bfloat16)
    asd_ref[...] = jnp.dot(h, att_ref[...], preferred_element_type=_f32)


def _tc_out_body(acc_ref, b_ref, w_ref, bo_ref, out_ref):
    hin = _combine(acc_ref, b_ref)
    out_ref[...] = (
        jnp.dot(hin, w_ref[...], preferred_element_type=_f32) + bo_ref[...])


def _full(shape):
    return pl.BlockSpec(shape, lambda i: tuple(0 for _ in shape))


def _tc_in(x_pad, w, att2):
    return pl.pallas_call(
        _tc_in_body,
        grid=(N2 // _RB,),
        in_specs=[
            pl.BlockSpec((_RB, x_pad.shape[1]), lambda i: (i, 0)),
            _full(w.shape),
            _full(att2.shape),
        ],
        out_specs=[
            pl.BlockSpec((_RB, HIDDEN), lambda i: (i, 0)),
            pl.BlockSpec((_RB, 2), lambda i: (i, 0)),
        ],
        out_shape=[
            jax.ShapeDtypeStruct((N2, HIDDEN), jnp.bfloat16),
            jax.ShapeDtypeStruct((N2, 2), _f32),
        ],
    )(x_pad, w, att2)


def _tc_mid(acc, b, w, att2):
    return pl.pallas_call(
        _tc_mid_body,
        grid=(N2 // _RB,),
        in_specs=[
            pl.BlockSpec((2, _RB, W80), lambda i: (0, i, 0)),
            _full(b.shape),
            _full(w.shape),
            _full(att2.shape),
        ],
        out_specs=[
            pl.BlockSpec((_RB, HIDDEN), lambda i: (i, 0)),
            pl.BlockSpec((_RB, 2), lambda i: (i, 0)),
        ],
        out_shape=[
            jax.ShapeDtypeStruct((N2, HIDDEN), jnp.bfloat16),
            jax.ShapeDtypeStruct((N2, 2), _f32),
        ],
    )(acc, b, w, att2)


def _tc_out(acc, b, w, bo):
    return pl.pallas_call(
        _tc_out_body,
        grid=(N2 // _RB,),
        in_specs=[
            pl.BlockSpec((2, _RB, W80), lambda i: (0, i, 0)),
            _full(b.shape),
            _full(w.shape),
            _full(bo.shape),
        ],
        out_specs=pl.BlockSpec((_RB, NCLASS), lambda i: (i, 0)),
        out_shape=jax.ShapeDtypeStruct((N2, NCLASS), _f32),
    )(acc, b, w, bo)


_QCOLS = tuple(
    32 * (j // 32)
    + (2 * (j % 32) if j % 32 < 16 else 2 * (j % 32 - 16) + 1)
    for j in range(HIDDEN))


@jax.jit
def kernel(x, edge_index, W1, a_s1, a_d1, b1, W2, a_s2, a_d2, b2, W_out, b_out):
    loops = jnp.arange(N, dtype=_i32)
    src = jnp.concatenate(
        [edge_index[0].astype(_i32), loops, jnp.zeros((E2 - E1,), _i32)])
    pad_dst = N + jnp.arange(E2 - E1, dtype=_i32) % (N2 - N)
    dst = jnp.concatenate([edge_index[1].astype(_i32), loops, pad_dst])
    eidx = ((dst << 16) | src).reshape(TOTB, K)

    x_pad = jnp.pad(x, ((0, N2 - N), (0, 0)))
    zn = jnp.zeros((ZR, W80), _f32)

    att1 = jnp.stack([a_s1, a_d1], axis=1)
    att2 = jnp.stack([a_s2, a_d2], axis=1)

    qc = jnp.array(_QCOLS, dtype=_i32)
    h1, asd1 = _tc_in(x_pad, W1, att1)
    acc1 = _edge_sc(eidx, asd1[:, 0], asd1[:, 1], h1, zn)
    h2, asd2 = _tc_mid(acc1, b1[qc].reshape(1, HIDDEN), W2[qc], att2)
    acc2 = _edge_sc(eidx, asd2[:, 0], asd2[:, 1], h2, zn)
    out = _tc_out(acc2, b2[qc].reshape(1, HIDDEN), W_out[qc],
                  b_out.reshape(1, NCLASS))
    return out[:N]
